# baseline (device time: 70015 ns/iter reference)
import jax
import jax.numpy as jnp
from jax import lax
from jax.experimental import pallas as pl
from jax.experimental.pallas import tpu as pltpu

B, H, D, BS = 16, 16, 64, 16
NB = 128


def kernel(Q, K, V, bt, lens):
    P_loc = K.shape[0]
    KV = P_loc * BS
    scale = D ** -0.5

    Qs = Q.reshape(B, H, D)
    Ks = K.reshape(KV, H, D)
    Vs = V.reshape(KV, H, D)
    lens2 = lens.reshape(B, 1)

    def body(q_ref, k_ref, v_ref, bt_ref, lens_ref, out_ref,
             o_send, ml_send, o_recv, ml_recv, send_sems, recv_sems):
        my_x = lax.axis_index("x")
        my_y = lax.axis_index("y")
        my_z = lax.axis_index("z")
        partner = (1 - my_x, my_y, my_z)

        barrier = pltpu.get_barrier_semaphore()
        pl.semaphore_signal(barrier, inc=1, device_id=partner,
                            device_id_type=pl.DeviceIdType.MESH)
        pl.semaphore_wait(barrier, 1)

        page_base = my_x * P_loc

        p_iota = lax.broadcasted_iota(jnp.int32, (B, P_loc, NB), 1)
        j_iota = lax.broadcasted_iota(jnp.int32, (B, P_loc, NB), 2)
        btb = bt_ref[:, :][:, None, :]
        lensb = lens_ref[:, :][:, :, None]
        match = (btb == p_iota + page_base) & (j_iota < lensb)
        c = jnp.sum(match.astype(jnp.float32), axis=2)

        c_keys = jnp.broadcast_to(c[:, :, None], (B, P_loc, BS)).reshape(B, KV)
        occ_keys = c_keys > 0.0

        ms, ls = [], []
        for h in range(H):
            q_h = q_ref[:, h, :]
            k_h = k_ref[:, h, :]
            s = lax.dot_general(
                q_h, k_h, (((1,), (1,)), ((), ())),
                preferred_element_type=jnp.float32) * scale
            s = jnp.where(occ_keys, s, -1e30)
            m = jnp.max(s, axis=1, keepdims=True)
            e = jnp.exp(s - m) * c_keys
            l = jnp.sum(e, axis=1, keepdims=True)
            v_h = v_ref[:, h, :]
            o = lax.dot_general(
                e, v_h, (((1,), (0,)), ((), ())),
                preferred_element_type=jnp.float32)
            o_send[:, h, :] = o
            ms.append(m)
            ls.append(l)
        ml_send[0, :, :] = jnp.concatenate(ms, axis=1)
        ml_send[1, :, :] = jnp.concatenate(ls, axis=1)

        rdma_o = pltpu.make_async_remote_copy(
            src_ref=o_send, dst_ref=o_recv,
            send_sem=send_sems.at[0], recv_sem=recv_sems.at[0],
            device_id=partner, device_id_type=pl.DeviceIdType.MESH)
        rdma_ml = pltpu.make_async_remote_copy(
            src_ref=ml_send, dst_ref=ml_recv,
            send_sem=send_sems.at[1], recv_sem=recv_sems.at[1],
            device_id=partner, device_id_type=pl.DeviceIdType.MESH)
        rdma_o.start()
        rdma_ml.start()
        rdma_o.wait()
        rdma_ml.wait()

        m_loc = ml_send[0, :, :]
        l_loc = ml_send[1, :, :]
        m_rem = ml_recv[0, :, :]
        l_rem = ml_recv[1, :, :]
        m_new = jnp.maximum(m_loc, m_rem)
        a_loc = jnp.exp(m_loc - m_new)
        a_rem = jnp.exp(m_rem - m_new)
        l_tot = a_loc * l_loc + a_rem * l_rem
        o_tot = (a_loc[:, :, None] * o_send[:, :, :]
                 + a_rem[:, :, None] * o_recv[:, :, :])
        out_ref[:, :, :] = o_tot / l_tot[:, :, None]

    out = pl.pallas_call(
        body,
        out_shape=jax.ShapeDtypeStruct((B, H, D), jnp.float32),
        in_specs=[pl.BlockSpec(memory_space=pltpu.VMEM)] * 5,
        out_specs=pl.BlockSpec(memory_space=pltpu.VMEM),
        scratch_shapes=[
            pltpu.VMEM((B, H, D), jnp.float32),
            pltpu.VMEM((2, B, H), jnp.float32),
            pltpu.VMEM((B, H, D), jnp.float32),
            pltpu.VMEM((2, B, H), jnp.float32),
            pltpu.SemaphoreType.DMA((2,)),
            pltpu.SemaphoreType.DMA((2,)),
        ],
        compiler_params=pltpu.CompilerParams(collective_id=0),
    )(Qs, Ks, Vs, bt, lens2)

    return out.reshape(B, 1, H, D)


# device time: 32505 ns/iter; 2.1540x vs baseline; 2.1540x over previous
import jax
import jax.numpy as jnp
from jax import lax
from jax.experimental import pallas as pl
from jax.experimental.pallas import tpu as pltpu

B, H, D, BS = 16, 16, 64, 16
NB = 128


def kernel(Q, K, V, bt, lens):
    P_loc = K.shape[0]
    KV = P_loc * BS
    scale = D ** -0.5

    QT = Q.reshape(B, H, D).transpose(1, 0, 2).astype(jnp.bfloat16)
    KT = K.reshape(KV, H, D).transpose(1, 0, 2).astype(jnp.bfloat16)
    VT = V.reshape(KV, H, D).transpose(1, 0, 2).astype(jnp.bfloat16)
    lens2 = lens.reshape(B, 1)

    def body(q_ref, k_ref, v_ref, bt_ref, lens_ref, out_ref,
             o_send, ml_send, o_recv, ml_recv, send_sems, recv_sems):
        my_x = lax.axis_index("x")
        my_y = lax.axis_index("y")
        my_z = lax.axis_index("z")
        partner = (1 - my_x, my_y, my_z)

        barrier = pltpu.get_barrier_semaphore()
        pl.semaphore_signal(barrier, inc=1, device_id=partner,
                            device_id_type=pl.DeviceIdType.MESH)
        pl.semaphore_wait(barrier, 1)

        page_base = my_x * P_loc

        p_iota = lax.broadcasted_iota(jnp.int32, (B, P_loc, NB), 1)
        j_iota = lax.broadcasted_iota(jnp.int32, (B, P_loc, NB), 2)
        btb = bt_ref[:, :][:, None, :]
        lensb = lens_ref[:, :][:, :, None]
        match = (btb == p_iota + page_base) & (j_iota < lensb)
        c = jnp.sum(match.astype(jnp.float32), axis=2)

        c_keys = jnp.broadcast_to(c[:, :, None], (B, P_loc, BS)).reshape(B, KV)
        occ_keys = c_keys > 0.0

        ms, ls = [], []
        for h in range(H):
            q_h = q_ref[h]
            k_h = k_ref[h]
            v_h = v_ref[h]
            s = lax.dot_general(
                q_h, k_h, (((1,), (1,)), ((), ())),
                preferred_element_type=jnp.float32) * scale
            s = jnp.where(occ_keys, s, -1e30)
            m = jnp.max(s, axis=1, keepdims=True)
            e = jnp.exp(s - m) * c_keys
            l = jnp.sum(e, axis=1, keepdims=True)
            o = lax.dot_general(
                e.astype(jnp.bfloat16), v_h, (((1,), (0,)), ((), ())),
                preferred_element_type=jnp.float32)
            o_send[h] = o
            ms.append(m)
            ls.append(l)
        ml_send[0, :, :] = jnp.concatenate(ms, axis=1)
        ml_send[1, :, :] = jnp.concatenate(ls, axis=1)

        rdma_o = pltpu.make_async_remote_copy(
            src_ref=o_send, dst_ref=o_recv,
            send_sem=send_sems.at[0], recv_sem=recv_sems.at[0],
            device_id=partner, device_id_type=pl.DeviceIdType.MESH)
        rdma_ml = pltpu.make_async_remote_copy(
            src_ref=ml_send, dst_ref=ml_recv,
            send_sem=send_sems.at[1], recv_sem=recv_sems.at[1],
            device_id=partner, device_id_type=pl.DeviceIdType.MESH)
        rdma_o.start()
        rdma_ml.start()
        rdma_o.wait()
        rdma_ml.wait()

        m_loc = ml_send[0, :, :]
        l_loc = ml_send[1, :, :]
        m_rem = ml_recv[0, :, :]
        l_rem = ml_recv[1, :, :]
        m_new = jnp.maximum(m_loc, m_rem)
        a_loc = jnp.exp(m_loc - m_new)
        a_rem = jnp.exp(m_rem - m_new)
        l_tot = a_loc * l_loc + a_rem * l_rem
        g_loc = jnp.transpose(a_loc / l_tot)
        g_rem = jnp.transpose(a_rem / l_tot)
        out_ref[:, :, :] = (g_loc[:, :, None] * o_send[:, :, :]
                            + g_rem[:, :, None] * o_recv[:, :, :])

    out = pl.pallas_call(
        body,
        out_shape=jax.ShapeDtypeStruct((H, B, D), jnp.float32),
        in_specs=[pl.BlockSpec(memory_space=pltpu.VMEM)] * 5,
        out_specs=pl.BlockSpec(memory_space=pltpu.VMEM),
        scratch_shapes=[
            pltpu.VMEM((H, B, D), jnp.float32),
            pltpu.VMEM((2, B, H), jnp.float32),
            pltpu.VMEM((H, B, D), jnp.float32),
            pltpu.VMEM((2, B, H), jnp.float32),
            pltpu.SemaphoreType.DMA((2,)),
            pltpu.SemaphoreType.DMA((2,)),
        ],
        compiler_params=pltpu.CompilerParams(collective_id=0),
    )(QT, KT, VT, bt, lens2)

    return out.transpose(1, 0, 2).reshape(B, 1, H, D)
